# baseline (speedup 1.0000x reference)
import functools

import jax
import jax.numpy as jnp
from jax import lax
from jax.experimental import pallas as pl
from jax.experimental.pallas import tpu as pltpu
from jax.experimental.pallas import tpu_sc as plsc

AXIAL_COLS = 64
HALF = 512
NUM_CORES = 2
NUM_SUBCORES = 16
NUM_WORKERS = NUM_CORES * NUM_SUBCORES
LANES = 16


def kernel(input_ids, row_emb, col_emb):
    seq = input_ids.shape[1]
    num_blocks = seq // AXIAL_COLS
    blocks_per_w = num_blocks // NUM_WORKERS

    rows_per_w = blocks_per_w * AXIAL_COLS

    mesh = plsc.VectorSubcoreMesh(core_axis_name="c", subcore_axis_name="s")

    @functools.partial(
        pl.kernel,
        mesh=mesh,
        out_type=jax.ShapeDtypeStruct((seq, 2 * HALF), jnp.float32),
        scratch_types=[
            pltpu.VMEM((blocks_per_w, HALF), jnp.float32),
            pltpu.VMEM((rows_per_w, HALF), jnp.float32),
            pltpu.VMEM_SHARED((AXIAL_COLS, HALF), jnp.float32),
            pltpu.SemaphoreType.DMA,
            pltpu.SemaphoreType.DMA,
        ],
    )
    def _axial(row_hbm, col_hbm, out_hbm, pair_v, rows_v, col_sp, sem_g, sem_w):
        sid = lax.axis_index("s")
        wid = sid * NUM_CORES + lax.axis_index("c")
        base = wid * rows_per_w
        seed = pltpu.async_copy(
            row_hbm.at[pl.ds(wid * blocks_per_w, blocks_per_w)], pair_v, sem_g
        )
        @pl.when(sid == 0)
        def _stage():
            pltpu.sync_copy(col_hbm, col_sp)

        plsc.subcore_barrier()
        ws = [
            pltpu.async_copy(
                col_sp,
                out_hbm.at[
                    pl.ds(base + j * AXIAL_COLS, AXIAL_COLS), pl.ds(HALF, HALF)
                ],
                sem_w,
            )
            for j in range(blocks_per_w)
        ]
        seed.wait()

        def body(k, carry):
            j = k // AXIAL_COLS
            for c in range(HALF // LANES):
                rows_v[k, pl.ds(c * LANES, LANES)] = pair_v[j, pl.ds(c * LANES, LANES)]
            return carry

        lax.fori_loop(0, rows_per_w, body, 0)
        ws.append(
            pltpu.async_copy(
                rows_v,
                out_hbm.at[pl.ds(base, rows_per_w), pl.ds(0, HALF)],
                sem_w,
            )
        )
        for w in ws:
            w.wait()

    return _axial(row_emb, col_emb)


# speedup vs baseline: 1.3838x; 1.3838x over previous
import functools

import jax
import jax.numpy as jnp
from jax import lax
from jax.experimental import pallas as pl
from jax.experimental.pallas import tpu as pltpu
from jax.experimental.pallas import tpu_sc as plsc

AXIAL_COLS = 64
HALF = 512
NUM_CORES = 2
NUM_SUBCORES = 16
NUM_WORKERS = NUM_CORES * NUM_SUBCORES
LANES = 16


def kernel(input_ids, row_emb, col_emb):
    seq = input_ids.shape[1]
    num_blocks = seq // AXIAL_COLS
    blocks_per_w = num_blocks // NUM_WORKERS

    rows_per_w = blocks_per_w * AXIAL_COLS

    mesh = plsc.VectorSubcoreMesh(core_axis_name="c", subcore_axis_name="s")

    @functools.partial(
        pl.kernel,
        mesh=mesh,
        out_type=jax.ShapeDtypeStruct((seq, 2 * HALF), jnp.float32),
        scratch_types=[
            pltpu.VMEM((blocks_per_w, HALF), jnp.float32),
            pltpu.VMEM((rows_per_w, HALF), jnp.float32),
            pltpu.VMEM_SHARED((AXIAL_COLS, HALF), jnp.float32),
            pltpu.SemaphoreType.DMA,
            pltpu.SemaphoreType.DMA,
        ],
    )
    def _axial(row_hbm, col_hbm, out_hbm, pair_v, rows_v, col_sp, sem_g, sem_w):
        sid = lax.axis_index("s")
        wid = sid * NUM_CORES + lax.axis_index("c")
        base = wid * rows_per_w
        seed = pltpu.async_copy(
            row_hbm.at[pl.ds(wid * blocks_per_w, blocks_per_w)], pair_v, sem_g
        )
        @pl.when(sid == 0)
        def _stage():
            pltpu.sync_copy(col_hbm, col_sp)

        plsc.subcore_barrier()
        ws = [
            pltpu.async_copy(
                col_sp,
                out_hbm.at[
                    pl.ds(base + j * AXIAL_COLS, AXIAL_COLS), pl.ds(HALF, HALF)
                ],
                sem_w,
            )
            for j in range(blocks_per_w)
        ]
        seed.wait()
        for j in range(blocks_per_w):
            vs = [pair_v[j, pl.ds(c * LANES, LANES)] for c in range(HALF // LANES)]

            def body(k, carry, j=j, vs=vs):
                for c in range(HALF // LANES):
                    rows_v[j * AXIAL_COLS + k, pl.ds(c * LANES, LANES)] = vs[c]
                return carry

            lax.fori_loop(0, AXIAL_COLS, body, 0)
            ws.append(
                pltpu.async_copy(
                    rows_v.at[pl.ds(j * AXIAL_COLS, AXIAL_COLS)],
                    out_hbm.at[pl.ds(base + j * AXIAL_COLS, AXIAL_COLS), pl.ds(0, HALF)],
                    sem_w,
                )
            )
        for w in ws:
            w.wait()

    return _axial(row_emb, col_emb)
